# baseline (speedup 1.0000x reference)
import functools

import jax
import jax.numpy as jnp
import numpy as np
from jax import lax
from jax.experimental import pallas as pl
from jax.experimental.pallas import tpu as pltpu
from jax.experimental.pallas import tpu_sc as plsc

_N = 10000
_E = 320000
_H = 128
_RBF = 16
_CUTOFF = 5.0
_LOG2 = float(np.log(2.0))
_LOG2E = float(np.log2(np.e))
_LN2 = float(np.log(2.0))
_EC = (1.0000072832543414, 0.6929312891618595, 0.24171026247088204,
       0.05166687743062185, 0.013676531087903416)
_LC = (8.116678753436612e-07, 1.442633690941714, -0.7202026916612381,
       0.4717218708231972, -0.3214835300848357, 0.18865272272011413,
       -0.0759208939439037, 0.014598605533668355)

_NC = 2
_NS = 16
_NW = _NC * _NS
_C = 64
_CHUNKS = _E // _C
_BCH = _CHUNKS // _NW
_XW = _CHUNKS % _NW
_RPT = 624
_TAIL = _N - _NS * _RPT


def _wf_body(ea_ref, ew_ref, w1_ref, b1_ref, w2_ref, b2_ref,
             o_ref, cut_ref):
    v = jnp.dot(ea_ref[...], w1_ref[...], preferred_element_type=jnp.float32)
    v = v + b1_ref[...]
    y = jnp.maximum(-jnp.abs(v) * _LOG2E, -120.0)
    yf = jnp.floor(y)
    fr = y - yf
    p = _EC[4]
    for c in (_EC[3], _EC[2], _EC[1], _EC[0]):
        p = p * fr + c
    scale = jax.lax.bitcast_convert_type(
        (yf.astype(jnp.int32) + 127) << 23, jnp.float32)
    t = p * scale
    q = _LC[7]
    for c in (_LC[6], _LC[5], _LC[4], _LC[3], _LC[2], _LC[1], _LC[0]):
        q = q * t + c
    v = jnp.maximum(v, 0.0) + q * _LN2 - _LOG2
    v = jnp.dot(v, w2_ref[...], preferred_element_type=jnp.float32) + b2_ref[...]
    o_ref[...] = v

    @pl.when(pl.program_id(0) == 0)
    def _():
        cut_ref[...] = 0.5 * (jnp.cos(ew_ref[...] * (np.pi / _CUTOFF)) + 1.0)


def _filters(edge_attr, edge_weight, fnet_w1, fnet_b1, fnet_w2, fnet_b2):
    be = 6400
    grid = (_E // be,)
    return pl.pallas_call(
        _wf_body,
        grid=grid,
        in_specs=[
            pl.BlockSpec((be, _RBF), lambda i: (i, 0)),
            pl.BlockSpec((_CHUNKS, _C), lambda i: (0, 0)),
            pl.BlockSpec((_RBF, _H), lambda i: (0, 0)),
            pl.BlockSpec((1, _H), lambda i: (0, 0)),
            pl.BlockSpec((_H, _H), lambda i: (0, 0)),
            pl.BlockSpec((1, _H), lambda i: (0, 0)),
        ],
        out_specs=[
            pl.BlockSpec((be, _H), lambda i: (i, 0)),
            pl.BlockSpec((_CHUNKS, _C), lambda i: (0, 0)),
        ],
        out_shape=[
            jax.ShapeDtypeStruct((_E, _H), jnp.float32),
            jax.ShapeDtypeStruct((_CHUNKS, _C), jnp.float32),
        ],
    )(edge_attr, edge_weight.reshape(_CHUNKS, _C), fnet_w1,
      fnet_b1.reshape(1, _H), fnet_w2, fnet_b2.reshape(1, _H))


def _h_body(x_ref, w_ref, o_ref):
    o_ref[...] = jnp.dot(x_ref[...], w_ref[...],
                         preferred_element_type=jnp.float32)


def _node_transform(x, lin1_w):
    bn = 2000
    return pl.pallas_call(
        _h_body,
        grid=(_N // bn,),
        in_specs=[
            pl.BlockSpec((bn, _H), lambda i: (i, 0)),
            pl.BlockSpec((_H, _H), lambda i: (0, 0)),
        ],
        out_specs=pl.BlockSpec((bn, _H), lambda i: (i, 0)),
        out_shape=jax.ShapeDtypeStruct((_N, _H), jnp.float32),
    )(x, lin1_w)


def _sc_body(h_hbm, wf_hbm, cut_hbm, ei_hbm, zero_hbm, out_hbm,
             src_r, dst_r, cut_r, rows_r, wf_r, agg_sh,
             sem_i, sem_g, sem_w, sem_s):
    cid = lax.axis_index("c")
    sid = lax.axis_index("s")
    wid = sid * _NC + cid
    ch0 = wid * _BCH + jnp.minimum(wid, _XW)
    nch = lax.select(wid < _XW, _BCH + 1, _BCH)

    pltpu.sync_copy(zero_hbm, rows_r.at[0])
    row0 = sid * _RPT
    done = 0
    while done < _RPT:
        ln = min(_C, _RPT - done)
        pltpu.sync_copy(rows_r.at[0, pl.ds(0, ln)],
                        agg_sh.at[pl.ds(row0 + done, ln)])
        done += ln

    @pl.when(sid == _NS - 1)
    def _():
        pltpu.sync_copy(rows_r.at[0, pl.ds(0, _TAIL)],
                        agg_sh.at[pl.ds(_NS * _RPT, _TAIL)])

    plsc.subcore_barrier()

    def issue_idx(k):
        b = lax.rem(k, 5)
        base = (ch0 + k) * _C
        pltpu.async_copy(ei_hbm.at[0, pl.ds(base, _C)], src_r.at[b], sem_i)
        pltpu.async_copy(ei_hbm.at[1, pl.ds(base, _C)], dst_r.at[b], sem_i)
        pltpu.async_copy(cut_hbm.at[ch0 + k], cut_r.at[b], sem_i)

    def wait_idx():
        pltpu.make_async_copy(ei_hbm.at[0, pl.ds(0, _C)], src_r.at[0],
                              sem_i).wait()
        pltpu.make_async_copy(ei_hbm.at[1, pl.ds(0, _C)], dst_r.at[0],
                              sem_i).wait()
        pltpu.make_async_copy(cut_hbm.at[0], cut_r.at[0], sem_i).wait()

    def issue_gather(k):
        pltpu.async_copy(h_hbm.at[src_r.at[lax.rem(k, 5)]],
                         rows_r.at[lax.rem(k, 3)], sem_g)

    def wait_gather():
        pltpu.make_async_copy(h_hbm.at[src_r.at[0]], rows_r.at[0],
                              sem_g).wait()

    def issue_wf(k):
        pltpu.async_copy(wf_hbm.at[pl.ds((ch0 + k) * _C, _C)],
                         wf_r.at[lax.rem(k, 2)], sem_w)

    def wait_wf():
        pltpu.make_async_copy(wf_hbm.at[pl.ds(0, _C)], wf_r.at[0],
                              sem_w).wait()

    def wait_scatter():
        pltpu.make_async_copy(rows_r.at[0], agg_sh.at[dst_r.at[0]],
                              sem_s).wait()

    def compute(b5, b3, b2):
        @plsc.parallel_loop(0, _C, unroll=4)
        def _(e):
            s = cut_r[b5, pl.ds(e, 16)][0]
            for j in range(_H // 16):
                sl = pl.ds(j * 16, 16)
                rows_r[b3, e, sl] = rows_r[b3, e, sl] * (wf_r[b2, e, sl] * s)

    issue_idx(0)
    issue_idx(1)
    issue_idx(2)
    issue_idx(3)
    wait_idx()
    wait_idx()
    issue_gather(0)
    issue_gather(1)
    issue_wf(0)
    issue_wf(1)

    def loop_body(i, carry):
        b5 = lax.rem(i, 5)
        b3 = lax.rem(i, 3)
        b2 = lax.rem(i, 2)
        wait_gather()
        wait_wf()
        compute(b5, b3, b2)
        pltpu.async_copy(rows_r.at[b3], agg_sh.at[dst_r.at[b5]],
                         sem_s, add=True)

        @pl.when(i > 0)
        def _():
            wait_scatter()

        @pl.when(i + 2 < nch)
        def _():
            wait_idx()
            issue_gather(i + 2)
            issue_wf(i + 2)

        @pl.when(i + 4 < nch)
        def _():
            issue_idx(i + 4)

        return carry

    lax.fori_loop(0, nch, loop_body, 0)
    wait_scatter()

    plsc.subcore_barrier()
    pltpu.sync_copy(agg_sh.at[pl.ds(row0, _RPT)],
                    out_hbm.at[cid, pl.ds(row0, _RPT)])

    @pl.when(sid == _NS - 1)
    def _():
        pltpu.sync_copy(agg_sh.at[pl.ds(_NS * _RPT, _TAIL)],
                        out_hbm.at[cid, pl.ds(_NS * _RPT, _TAIL)])


def _aggregate(h, wf, cut, edge_index):
    mesh = plsc.VectorSubcoreMesh(core_axis_name="c", subcore_axis_name="s")
    call = functools.partial(
        pl.kernel,
        out_type=jax.ShapeDtypeStruct((_NC, _N, _H), jnp.float32),
        mesh=mesh,
        scratch_types=[
            pltpu.VMEM((5, _C), jnp.int32),
            pltpu.VMEM((5, _C), jnp.int32),
            pltpu.VMEM((6, _C), jnp.float32),
            pltpu.VMEM((3, _C, _H), jnp.float32),
            pltpu.VMEM((2, _C, _H), jnp.float32),
            pltpu.VMEM_SHARED((_N, _H), jnp.float32),
            pltpu.SemaphoreType.DMA,
            pltpu.SemaphoreType.DMA,
            pltpu.SemaphoreType.DMA,
            pltpu.SemaphoreType.DMA,
        ],
    )(_sc_body)
    zero = jnp.zeros((_C, _H), jnp.float32)
    return call(h, wf, cut, edge_index, zero)


def _out_body(a_ref, w2_ref, b2_ref, w3_ref, b3_ref, o_ref):
    a = a_ref[0] + a_ref[1]
    t = jnp.dot(a, w2_ref[...], preferred_element_type=jnp.float32)
    t = jnp.tanh(t + b2_ref[...])
    o_ref[...] = jnp.dot(t, w3_ref[...],
                         preferred_element_type=jnp.float32) + b3_ref[...]


def _tail(agg2, lin2_w, lin2_b, lin_w, lin_b):
    bn = 2000
    return pl.pallas_call(
        _out_body,
        grid=(_N // bn,),
        in_specs=[
            pl.BlockSpec((_NC, bn, _H), lambda i: (0, i, 0)),
            pl.BlockSpec((_H, _H), lambda i: (0, 0)),
            pl.BlockSpec((1, _H), lambda i: (0, 0)),
            pl.BlockSpec((_H, _H), lambda i: (0, 0)),
            pl.BlockSpec((1, _H), lambda i: (0, 0)),
        ],
        out_specs=pl.BlockSpec((bn, _H), lambda i: (i, 0)),
        out_shape=jax.ShapeDtypeStruct((_N, _H), jnp.float32),
    )(agg2, lin2_w, lin2_b.reshape(1, _H), lin_w, lin_b.reshape(1, _H))


def kernel(x, edge_index, edge_weight, edge_attr, atom_types, seq_neighs,
           lin1_w, fnet_w1, fnet_b1, fnet_w2, fnet_b2, lin2_w, lin2_b,
           lin_w, lin_b):
    wf, cut = _filters(edge_attr, edge_weight, fnet_w1, fnet_b1,
                       fnet_w2, fnet_b2)
    h = _node_transform(x, lin1_w)
    agg2 = _aggregate(h, wf, cut, edge_index)
    return _tail(agg2, lin2_w, lin2_b, lin_w, lin_b)


# speedup vs baseline: 1.0946x; 1.0946x over previous
import functools

import jax
import jax.numpy as jnp
import numpy as np
from jax import lax
from jax.experimental import pallas as pl
from jax.experimental.pallas import tpu as pltpu
from jax.experimental.pallas import tpu_sc as plsc

_N = 10000
_E = 320000
_H = 128
_RBF = 16
_CUTOFF = 5.0
_LOG2 = float(np.log(2.0))
_LOG2E = float(np.log2(np.e))
_LN2 = float(np.log(2.0))
_EC = (1.0000072832543414, 0.6929312891618595, 0.24171026247088204,
       0.05166687743062185, 0.013676531087903416)
_LC = (8.116678753436612e-07, 1.442633690941714, -0.7202026916612381,
       0.4717218708231972, -0.3214835300848357, 0.18865272272011413,
       -0.0759208939439037, 0.014598605533668355)

_NC = 2
_NS = 16
_NW = _NC * _NS
_C = 64
_CHUNKS = _E // _C
_BCH = _CHUNKS // _NW
_XW = _CHUNKS % _NW
_RPT = 624
_TAIL = _N - _NS * _RPT


def _wf_body(ea_ref, ew_ref, ei_ref, w1_ref, b1_ref, w2_ref, b2_ref,
             o_ref, cut_ref, src_ref, dst_ref):
    v = jnp.dot(ea_ref[...], w1_ref[...], preferred_element_type=jnp.float32)
    v = v + b1_ref[...]
    v = jnp.maximum(v, 0.0) + jnp.log(1.0 + jnp.exp(-jnp.abs(v))) - _LOG2
    v = jnp.dot(v, w2_ref[...], preferred_element_type=jnp.float32) + b2_ref[...]
    o_ref[...] = v
    for q in range(src_ref.shape[0]):
        sl = pl.ds(q * _C, _C)
        src_ref[q, :] = ei_ref[0, sl]
        dst_ref[q, :] = ei_ref[1, sl]

    @pl.when(pl.program_id(0) == 0)
    def _():
        cut_ref[...] = 0.5 * (jnp.cos(ew_ref[...] * (np.pi / _CUTOFF)) + 1.0)


def _filters(edge_attr, edge_weight, edge_index, fnet_w1, fnet_b1,
             fnet_w2, fnet_b2):
    be = 2560
    grid = (_E // be,)
    return pl.pallas_call(
        _wf_body,
        grid=grid,
        in_specs=[
            pl.BlockSpec((be, _RBF), lambda i: (i, 0)),
            pl.BlockSpec((_CHUNKS, _C), lambda i: (0, 0)),
            pl.BlockSpec((2, be), lambda i: (0, i)),
            pl.BlockSpec((_RBF, _H), lambda i: (0, 0)),
            pl.BlockSpec((1, _H), lambda i: (0, 0)),
            pl.BlockSpec((_H, _H), lambda i: (0, 0)),
            pl.BlockSpec((1, _H), lambda i: (0, 0)),
        ],
        out_specs=[
            pl.BlockSpec((be, _H), lambda i: (i, 0)),
            pl.BlockSpec((_CHUNKS, _C), lambda i: (0, 0)),
            pl.BlockSpec((be // _C, _C), lambda i: (i, 0)),
            pl.BlockSpec((be // _C, _C), lambda i: (i, 0)),
        ],
        out_shape=[
            jax.ShapeDtypeStruct((_E, _H), jnp.float32),
            jax.ShapeDtypeStruct((_CHUNKS, _C), jnp.float32),
            jax.ShapeDtypeStruct((_CHUNKS, _C), jnp.int32),
            jax.ShapeDtypeStruct((_CHUNKS, _C), jnp.int32),
        ],
    )(edge_attr, edge_weight.reshape(_CHUNKS, _C), edge_index, fnet_w1,
      fnet_b1.reshape(1, _H), fnet_w2, fnet_b2.reshape(1, _H))


def _h_body(x_ref, w_ref, o_ref):
    o_ref[...] = jnp.dot(x_ref[...], w_ref[...],
                         preferred_element_type=jnp.float32)


def _node_transform(x, lin1_w):
    bn = 2000
    return pl.pallas_call(
        _h_body,
        grid=(_N // bn,),
        in_specs=[
            pl.BlockSpec((bn, _H), lambda i: (i, 0)),
            pl.BlockSpec((_H, _H), lambda i: (0, 0)),
        ],
        out_specs=pl.BlockSpec((bn, _H), lambda i: (i, 0)),
        out_shape=jax.ShapeDtypeStruct((_N, _H), jnp.float32),
    )(x, lin1_w)


def _sc_body(h_hbm, wf_hbm, cut_hbm, src_hbm, dst_hbm, zero_hbm, out_hbm,
             src_r, dst_r, cut_r, rows_r, wf_r, agg_sh,
             sem_i, sem_g, sem_w, sem_s):
    cid = lax.axis_index("c")
    sid = lax.axis_index("s")
    wid = sid * _NC + cid
    ch0 = wid * _BCH + jnp.minimum(wid, _XW)
    nch = lax.select(wid < _XW, _BCH + 1, _BCH)

    pltpu.sync_copy(zero_hbm, rows_r.at[0])
    row0 = sid * _RPT
    done = 0
    while done < _RPT:
        ln = min(_C, _RPT - done)
        pltpu.sync_copy(rows_r.at[0, pl.ds(0, ln)],
                        agg_sh.at[pl.ds(row0 + done, ln)])
        done += ln

    @pl.when(sid == _NS - 1)
    def _():
        pltpu.sync_copy(rows_r.at[0, pl.ds(0, _TAIL)],
                        agg_sh.at[pl.ds(_NS * _RPT, _TAIL)])

    plsc.subcore_barrier()

    def issue_idx(k):
        b = lax.rem(k, 5)
        pltpu.async_copy(src_hbm.at[ch0 + k], src_r.at[b], sem_i)
        pltpu.async_copy(dst_hbm.at[ch0 + k], dst_r.at[b], sem_i)
        pltpu.async_copy(cut_hbm.at[ch0 + k], cut_r.at[b], sem_i)

    def wait_idx():
        pltpu.make_async_copy(src_hbm.at[0], src_r.at[0], sem_i).wait()
        pltpu.make_async_copy(dst_hbm.at[0], dst_r.at[0], sem_i).wait()
        pltpu.make_async_copy(cut_hbm.at[0], cut_r.at[0], sem_i).wait()

    def issue_gather(k):
        pltpu.async_copy(h_hbm.at[src_r.at[lax.rem(k, 5)]],
                         rows_r.at[lax.rem(k, 3)], sem_g)

    def wait_gather():
        pltpu.make_async_copy(h_hbm.at[src_r.at[0]], rows_r.at[0],
                              sem_g).wait()

    def issue_wf(k):
        pltpu.async_copy(wf_hbm.at[pl.ds((ch0 + k) * _C, _C)],
                         wf_r.at[lax.rem(k, 2)], sem_w)

    def wait_wf():
        pltpu.make_async_copy(wf_hbm.at[pl.ds(0, _C)], wf_r.at[0],
                              sem_w).wait()

    def wait_scatter():
        pltpu.make_async_copy(rows_r.at[0], agg_sh.at[dst_r.at[0]],
                              sem_s).wait()

    def compute(b5, b3, b2):
        @plsc.parallel_loop(0, _C, unroll=4)
        def _(e):
            s = cut_r[b5, pl.ds(e, 16)][0]
            for j in range(_H // 16):
                sl = pl.ds(j * 16, 16)
                rows_r[b3, e, sl] = rows_r[b3, e, sl] * (wf_r[b2, e, sl] * s)

    issue_idx(0)
    issue_idx(1)
    issue_idx(2)
    issue_idx(3)
    wait_idx()
    wait_idx()
    issue_gather(0)
    issue_gather(1)
    issue_wf(0)
    issue_wf(1)

    def loop_body(i, carry):
        b5 = lax.rem(i, 5)
        b3 = lax.rem(i, 3)
        b2 = lax.rem(i, 2)
        wait_gather()
        wait_wf()
        compute(b5, b3, b2)
        pltpu.async_copy(rows_r.at[b3], agg_sh.at[dst_r.at[b5]],
                         sem_s, add=True)

        @pl.when(i > 0)
        def _():
            wait_scatter()

        @pl.when(i + 2 < nch)
        def _():
            wait_idx()
            issue_gather(i + 2)
            issue_wf(i + 2)

        @pl.when(i + 4 < nch)
        def _():
            issue_idx(i + 4)

        return carry

    lax.fori_loop(0, nch, loop_body, 0)
    wait_scatter()

    plsc.subcore_barrier()
    pltpu.sync_copy(agg_sh.at[pl.ds(row0, _RPT)],
                    out_hbm.at[cid, pl.ds(row0, _RPT)])

    @pl.when(sid == _NS - 1)
    def _():
        pltpu.sync_copy(agg_sh.at[pl.ds(_NS * _RPT, _TAIL)],
                        out_hbm.at[cid, pl.ds(_NS * _RPT, _TAIL)])


def _aggregate(h, wf, cut, src2, dst2):
    mesh = plsc.VectorSubcoreMesh(core_axis_name="c", subcore_axis_name="s")
    call = functools.partial(
        pl.kernel,
        out_type=jax.ShapeDtypeStruct((_NC, _N, _H), jnp.float32),
        mesh=mesh,
        scratch_types=[
            pltpu.VMEM((5, _C), jnp.int32),
            pltpu.VMEM((5, _C), jnp.int32),
            pltpu.VMEM((6, _C), jnp.float32),
            pltpu.VMEM((3, _C, _H), jnp.float32),
            pltpu.VMEM((2, _C, _H), jnp.float32),
            pltpu.VMEM_SHARED((_N, _H), jnp.float32),
            pltpu.SemaphoreType.DMA,
            pltpu.SemaphoreType.DMA,
            pltpu.SemaphoreType.DMA,
            pltpu.SemaphoreType.DMA,
        ],
    )(_sc_body)
    zero = jnp.zeros((_C, _H), jnp.float32)
    return call(h, wf, cut, src2, dst2, zero)


def _out_body(a_ref, w2_ref, b2_ref, w3_ref, b3_ref, o_ref):
    a = a_ref[0] + a_ref[1]
    t = jnp.dot(a, w2_ref[...], preferred_element_type=jnp.float32)
    t = jnp.tanh(t + b2_ref[...])
    o_ref[...] = jnp.dot(t, w3_ref[...],
                         preferred_element_type=jnp.float32) + b3_ref[...]


def _tail(agg2, lin2_w, lin2_b, lin_w, lin_b):
    bn = 2000
    return pl.pallas_call(
        _out_body,
        grid=(_N // bn,),
        in_specs=[
            pl.BlockSpec((_NC, bn, _H), lambda i: (0, i, 0)),
            pl.BlockSpec((_H, _H), lambda i: (0, 0)),
            pl.BlockSpec((1, _H), lambda i: (0, 0)),
            pl.BlockSpec((_H, _H), lambda i: (0, 0)),
            pl.BlockSpec((1, _H), lambda i: (0, 0)),
        ],
        out_specs=pl.BlockSpec((bn, _H), lambda i: (i, 0)),
        out_shape=jax.ShapeDtypeStruct((_N, _H), jnp.float32),
    )(agg2, lin2_w, lin2_b.reshape(1, _H), lin_w, lin_b.reshape(1, _H))


def kernel(x, edge_index, edge_weight, edge_attr, atom_types, seq_neighs,
           lin1_w, fnet_w1, fnet_b1, fnet_w2, fnet_b2, lin2_w, lin2_b,
           lin_w, lin_b):
    wf, cut, src2, dst2 = _filters(edge_attr, edge_weight, edge_index,
                                   fnet_w1, fnet_b1, fnet_w2, fnet_b2)
    h = _node_transform(x, lin1_w)
    agg2 = _aggregate(h, wf, cut, src2, dst2)
    return _tail(agg2, lin2_w, lin2_b, lin_w, lin_b)


# speedup vs baseline: 1.6473x; 1.5050x over previous
import functools

import jax
import jax.numpy as jnp
import numpy as np
from jax import lax
from jax.experimental import pallas as pl
from jax.experimental.pallas import tpu as pltpu
from jax.experimental.pallas import tpu_sc as plsc

_N = 10000
_E = 320000
_H = 128
_RBF = 16
_CUTOFF = 5.0
_LOG2 = float(np.log(2.0))
_LOG2E = float(np.log2(np.e))
_LN2 = float(np.log(2.0))
_EC = (1.0000072832543414, 0.6929312891618595, 0.24171026247088204,
       0.05166687743062185, 0.013676531087903416)
_LC = (8.116678753436612e-07, 1.442633690941714, -0.7202026916612381,
       0.4717218708231972, -0.3214835300848357, 0.18865272272011413,
       -0.0759208939439037, 0.014598605533668355)

_NC = 2
_NS = 16
_NW = _NC * _NS
_C = 64
_CHUNKS = _E // _C
_BCH = _CHUNKS // _NW
_XW = _CHUNKS % _NW
_RPT = 624
_TAIL = _N - _NS * _RPT


def _wf_body(ea_ref, ew_ref, w1_ref, b1_ref, w2_ref, b2_ref,
             o_ref, cut_ref):
    v = lax.dot_general(ea_ref[...], w1_ref[...],
                        (((0,), (0,)), ((), ())),
                        preferred_element_type=jnp.float32)
    v = v + b1_ref[...]
    v = jnp.maximum(v, 0.0) + jnp.log(1.0 + jnp.exp(-jnp.abs(v))) - _LOG2
    v = jnp.dot(v, w2_ref[...], preferred_element_type=jnp.float32) + b2_ref[...]
    o_ref[...] = v

    @pl.when(pl.program_id(0) == 0)
    def _():
        cut_ref[...] = 0.5 * (jnp.cos(ew_ref[...] * (np.pi / _CUTOFF)) + 1.0)


def _filters(edge_attr, edge_weight, fnet_w1, fnet_b1, fnet_w2, fnet_b2):
    be = 6400
    grid = (_E // be,)
    return pl.pallas_call(
        _wf_body,
        grid=grid,
        in_specs=[
            pl.BlockSpec((_RBF, be), lambda i: (0, i)),
            pl.BlockSpec((_CHUNKS, _C), lambda i: (0, 0)),
            pl.BlockSpec((_RBF, _H), lambda i: (0, 0)),
            pl.BlockSpec((1, _H), lambda i: (0, 0)),
            pl.BlockSpec((_H, _H), lambda i: (0, 0)),
            pl.BlockSpec((1, _H), lambda i: (0, 0)),
        ],
        out_specs=[
            pl.BlockSpec((be, _H), lambda i: (i, 0)),
            pl.BlockSpec((_CHUNKS, _C), lambda i: (0, 0)),
        ],
        out_shape=[
            jax.ShapeDtypeStruct((_E, _H), jnp.float32),
            jax.ShapeDtypeStruct((_CHUNKS, _C), jnp.float32),
        ],
    )(edge_attr.T, edge_weight.reshape(_CHUNKS, _C), fnet_w1,
      fnet_b1.reshape(1, _H), fnet_w2, fnet_b2.reshape(1, _H))


def _h_body(x_ref, w_ref, o_ref):
    o_ref[...] = jnp.dot(x_ref[...], w_ref[...],
                         preferred_element_type=jnp.float32)


def _node_transform(x, lin1_w):
    bn = 2000
    return pl.pallas_call(
        _h_body,
        grid=(_N // bn,),
        in_specs=[
            pl.BlockSpec((bn, _H), lambda i: (i, 0)),
            pl.BlockSpec((_H, _H), lambda i: (0, 0)),
        ],
        out_specs=pl.BlockSpec((bn, _H), lambda i: (i, 0)),
        out_shape=jax.ShapeDtypeStruct((_N, _H), jnp.float32),
    )(x, lin1_w)


def _sc_body(h_hbm, wf_hbm, cut_hbm, ei_hbm, zero_hbm, out_hbm,
             src_r, dst_r, cut_r, rows_r, wf_r, agg_sh,
             sem_i, sem_g, sem_w, sem_s):
    cid = lax.axis_index("c")
    sid = lax.axis_index("s")
    wid = sid * _NC + cid
    ch0 = wid * _BCH + jnp.minimum(wid, _XW)
    nch = lax.select(wid < _XW, _BCH + 1, _BCH)

    pltpu.sync_copy(zero_hbm, rows_r.at[0])
    row0 = sid * _RPT
    done = 0
    while done < _RPT:
        ln = min(_C, _RPT - done)
        pltpu.sync_copy(rows_r.at[0, pl.ds(0, ln)],
                        agg_sh.at[pl.ds(row0 + done, ln)])
        done += ln

    @pl.when(sid == _NS - 1)
    def _():
        pltpu.sync_copy(rows_r.at[0, pl.ds(0, _TAIL)],
                        agg_sh.at[pl.ds(_NS * _RPT, _TAIL)])

    plsc.subcore_barrier()

    def issue_idx(k):
        b = lax.rem(k, 5)
        base = (ch0 + k) * _C
        pltpu.async_copy(ei_hbm.at[0, pl.ds(base, _C)], src_r.at[b], sem_i)
        pltpu.async_copy(ei_hbm.at[1, pl.ds(base, _C)], dst_r.at[b], sem_i)
        pltpu.async_copy(cut_hbm.at[ch0 + k], cut_r.at[b], sem_i)

    def wait_idx():
        pltpu.make_async_copy(ei_hbm.at[0, pl.ds(0, _C)], src_r.at[0],
                              sem_i).wait()
        pltpu.make_async_copy(ei_hbm.at[1, pl.ds(0, _C)], dst_r.at[0],
                              sem_i).wait()
        pltpu.make_async_copy(cut_hbm.at[0], cut_r.at[0], sem_i).wait()

    def issue_gather(k):
        pltpu.async_copy(h_hbm.at[src_r.at[lax.rem(k, 5)]],
                         rows_r.at[lax.rem(k, 3)], sem_g)

    def wait_gather():
        pltpu.make_async_copy(h_hbm.at[src_r.at[0]], rows_r.at[0],
                              sem_g).wait()

    def issue_wf(k):
        pltpu.async_copy(wf_hbm.at[pl.ds((ch0 + k) * _C, _C)],
                         wf_r.at[lax.rem(k, 2)], sem_w)

    def wait_wf():
        pltpu.make_async_copy(wf_hbm.at[pl.ds(0, _C)], wf_r.at[0],
                              sem_w).wait()

    def wait_scatter():
        pltpu.make_async_copy(rows_r.at[0], agg_sh.at[dst_r.at[0]],
                              sem_s).wait()

    def compute(b5, b3, b2):
        @plsc.parallel_loop(0, _C, unroll=4)
        def _(e):
            s = cut_r[b5, pl.ds(e, 16)][0]
            for j in range(_H // 16):
                sl = pl.ds(j * 16, 16)
                rows_r[b3, e, sl] = rows_r[b3, e, sl] * (wf_r[b2, e, sl] * s)

    issue_idx(0)
    issue_idx(1)
    issue_idx(2)
    issue_idx(3)
    wait_idx()
    wait_idx()
    issue_gather(0)
    issue_gather(1)
    issue_wf(0)
    issue_wf(1)

    def loop_body(i, carry):
        b5 = lax.rem(i, 5)
        b3 = lax.rem(i, 3)
        b2 = lax.rem(i, 2)
        wait_gather()
        wait_wf()
        compute(b5, b3, b2)
        pltpu.async_copy(rows_r.at[b3], agg_sh.at[dst_r.at[b5]],
                         sem_s, add=True)

        @pl.when(i > 0)
        def _():
            wait_scatter()

        @pl.when(i + 2 < nch)
        def _():
            wait_idx()
            issue_gather(i + 2)
            issue_wf(i + 2)

        @pl.when(i + 4 < nch)
        def _():
            issue_idx(i + 4)

        return carry

    lax.fori_loop(0, nch, loop_body, 0)
    wait_scatter()

    plsc.subcore_barrier()
    pltpu.sync_copy(agg_sh.at[pl.ds(row0, _RPT)],
                    out_hbm.at[cid, pl.ds(row0, _RPT)])

    @pl.when(sid == _NS - 1)
    def _():
        pltpu.sync_copy(agg_sh.at[pl.ds(_NS * _RPT, _TAIL)],
                        out_hbm.at[cid, pl.ds(_NS * _RPT, _TAIL)])


def _aggregate(h, wf, cut, edge_index):
    mesh = plsc.VectorSubcoreMesh(core_axis_name="c", subcore_axis_name="s")
    call = functools.partial(
        pl.kernel,
        out_type=jax.ShapeDtypeStruct((_NC, _N, _H), jnp.float32),
        mesh=mesh,
        scratch_types=[
            pltpu.VMEM((5, _C), jnp.int32),
            pltpu.VMEM((5, _C), jnp.int32),
            pltpu.VMEM((6, _C), jnp.float32),
            pltpu.VMEM((3, _C, _H), jnp.float32),
            pltpu.VMEM((2, _C, _H), jnp.float32),
            pltpu.VMEM_SHARED((_N, _H), jnp.float32),
            pltpu.SemaphoreType.DMA,
            pltpu.SemaphoreType.DMA,
            pltpu.SemaphoreType.DMA,
            pltpu.SemaphoreType.DMA,
        ],
    )(_sc_body)
    zero = jnp.zeros((_C, _H), jnp.float32)
    return call(h, wf, cut, edge_index, zero)


def _out_body(a_ref, w2_ref, b2_ref, w3_ref, b3_ref, o_ref):
    a = a_ref[0] + a_ref[1]
    t = jnp.dot(a, w2_ref[...], preferred_element_type=jnp.float32)
    t = jnp.tanh(t + b2_ref[...])
    o_ref[...] = jnp.dot(t, w3_ref[...],
                         preferred_element_type=jnp.float32) + b3_ref[...]


def _tail(agg2, lin2_w, lin2_b, lin_w, lin_b):
    bn = 2000
    return pl.pallas_call(
        _out_body,
        grid=(_N // bn,),
        in_specs=[
            pl.BlockSpec((_NC, bn, _H), lambda i: (0, i, 0)),
            pl.BlockSpec((_H, _H), lambda i: (0, 0)),
            pl.BlockSpec((1, _H), lambda i: (0, 0)),
            pl.BlockSpec((_H, _H), lambda i: (0, 0)),
            pl.BlockSpec((1, _H), lambda i: (0, 0)),
        ],
        out_specs=pl.BlockSpec((bn, _H), lambda i: (i, 0)),
        out_shape=jax.ShapeDtypeStruct((_N, _H), jnp.float32),
    )(agg2, lin2_w, lin2_b.reshape(1, _H), lin_w, lin_b.reshape(1, _H))


def kernel(x, edge_index, edge_weight, edge_attr, atom_types, seq_neighs,
           lin1_w, fnet_w1, fnet_b1, fnet_w2, fnet_b2, lin2_w, lin2_b,
           lin_w, lin_b):
    wf, cut = _filters(edge_attr, edge_weight, fnet_w1, fnet_b1,
                       fnet_w2, fnet_b2)
    h = _node_transform(x, lin1_w)
    agg2 = _aggregate(h, wf, cut, edge_index)
    return _tail(agg2, lin2_w, lin2_b, lin_w, lin_b)
